# baseline (speedup 1.0000x reference)
import functools

import jax
import jax.numpy as jnp
from jax import lax
from jax.experimental import pallas as pl
from jax.experimental.pallas import tpu as pltpu
from jax.experimental.pallas import tpu_sc as plsc

_EPS = 1e-5

_NC = 2
_NS = 16
_NW = _NC * _NS
_C = 96
_CC = 128


def _pad_dst(n):
    q = _NS * 8
    return ((n + q - 1) // q) * q


def _chunks(total, step):
    out = []
    o = 0
    while o < total:
        s = min(step, total - o)
        out.append((o, s))
        o += s
    return out


def _seg_sum_body(n_pad, n_chunk, h, x_hbm, src_hbm, dst_hbm, out_hbm,
                  src_v, dst_v, rows_v, zero_v, acc_sh, sem):
    cid = lax.axis_index("c")
    sid = lax.axis_index("s")
    wid = cid * _NS + sid
    rows_per_sub = n_pad // _NS

    @pl.loop(0, 8)
    def _(i):
        for c in range(h // 16):
            zero_v[i, pl.ds(c * 16, 16)] = jnp.zeros((16,), jnp.float32)

    for off, sz in _chunks(rows_per_sub, 8):
        pltpu.sync_copy(zero_v.at[pl.ds(0, sz)],
                        acc_sh.at[pl.ds(sid * rows_per_sub + off, sz)])

    pltpu.sync_copy(src_hbm.at[wid], src_v)
    pltpu.sync_copy(dst_hbm.at[wid], dst_v)
    plsc.subcore_barrier()

    @pl.loop(0, n_chunk)
    def _(j):
        pltpu.async_copy(x_hbm.at[src_v.at[j]], rows_v, sem).wait()
        pltpu.sync_copy(rows_v, acc_sh.at[dst_v.at[j]], add=True)

    plsc.subcore_barrier()

    for off, sz in _chunks(rows_per_sub, 128):
        base = sid * rows_per_sub + off
        pltpu.sync_copy(acc_sh.at[pl.ds(base, sz)],
                        out_hbm.at[cid].at[pl.ds(base, sz)])


def _make_seg_sum(n_pad, n_chunk, h):
    mesh = plsc.VectorSubcoreMesh(core_axis_name="c", subcore_axis_name="s")
    return pl.kernel(
        functools.partial(_seg_sum_body, n_pad, n_chunk, h),
        out_type=jax.ShapeDtypeStruct((_NC, n_pad, h), jnp.float32),
        mesh=mesh,
        scratch_types=[
            pltpu.VMEM((n_chunk, _C), jnp.int32),
            pltpu.VMEM((n_chunk, _C), jnp.int32),
            pltpu.VMEM((_C, h), jnp.float32),
            pltpu.VMEM((8, h), jnp.float32),
            pltpu.VMEM_SHARED((n_pad, h), jnp.float32),
            pltpu.SemaphoreType.DMA,
        ],
    )


def _counts_body(n_pad, n_chunk, dst_hbm, cnt_hbm, dst_v, ones_v, acc_sh):
    cid = lax.axis_index("c")
    sid = lax.axis_index("s")
    wid = cid * _NS + sid
    rows_per_sub = n_pad // _NS

    @pl.loop(0, _CC)
    def _(i):
        for c in range(8):
            ones_v[i, pl.ds(c * 16, 16)] = jnp.zeros((16,), jnp.float32)

    for off, sz in _chunks(rows_per_sub, _CC):
        pltpu.sync_copy(ones_v.at[pl.ds(0, sz)],
                        acc_sh.at[pl.ds(sid * rows_per_sub + off, sz)])

    @pl.loop(0, _CC)
    def _(i):
        for c in range(8):
            ones_v[i, pl.ds(c * 16, 16)] = jnp.ones((16,), jnp.float32)

    pltpu.sync_copy(dst_hbm.at[wid], dst_v)
    plsc.subcore_barrier()

    @pl.loop(0, n_chunk)
    def _(j):
        pltpu.sync_copy(ones_v, acc_sh.at[dst_v.at[j]], add=True)

    plsc.subcore_barrier()

    for off, sz in _chunks(rows_per_sub, 128):
        base = sid * rows_per_sub + off
        pltpu.sync_copy(acc_sh.at[pl.ds(base, sz)],
                        cnt_hbm.at[cid].at[pl.ds(base, sz)])


def _make_counts(n_pad, n_chunk):
    mesh = plsc.VectorSubcoreMesh(core_axis_name="c", subcore_axis_name="s")
    return pl.kernel(
        functools.partial(_counts_body, n_pad, n_chunk),
        out_type=jax.ShapeDtypeStruct((_NC, n_pad, 128), jnp.float32),
        mesh=mesh,
        scratch_types=[
            pltpu.VMEM((n_chunk, _CC), jnp.int32),
            pltpu.VMEM((_CC, 128), jnp.float32),
            pltpu.VMEM_SHARED((n_pad, 128), jnp.float32),
        ],
    )


def _bn_relu(z, g, b):
    mu = jnp.mean(z, axis=0, keepdims=True)
    var = jnp.mean((z - mu) ** 2, axis=0, keepdims=True)
    return jnp.maximum((z - mu) / jnp.sqrt(var + _EPS) * g + b, 0.0)


def _prep_body(xu, xi, Wu, bu, gu, bbu, Wi, bi, gi, bbi, hu_o, hi_o):
    zu = jnp.dot(xu[...], Wu[...], preferred_element_type=jnp.float32) + bu[...]
    hu_o[...] = _bn_relu(zu, gu[...], bbu[...])
    zi = jnp.dot(xi[...], Wi[...], preferred_element_type=jnp.float32) + bi[...]
    hi_o[...] = _bn_relu(zi, gi[...], bbi[...])


def _layer_body(P, cnt, h, Wl, bl, Wr, g, bb, out):
    n = h.shape[0]
    agg = (P[0, :n] + P[1, :n]) / jnp.maximum(cnt[:n, :1], 1.0)
    z = (jnp.dot(agg, Wl[...], preferred_element_type=jnp.float32)
         + bl[...]
         + jnp.dot(h[...], Wr[...], preferred_element_type=jnp.float32))
    out[...] = _bn_relu(z, g[...], bb[...]) + h[...]


def _final_body(hu, hi, W, b, ou, oi):
    ou[...] = jnp.dot(hu[...], W[...], preferred_element_type=jnp.float32) + b[...]
    oi[...] = jnp.dot(hi[...], W[...], preferred_element_type=jnp.float32) + b[...]


def _r2(v):
    return v.reshape(1, -1)


def kernel(x_user, x_item, edge_ui, edge_iu, params):
    p = params
    N_U, H = x_user.shape
    N_I = x_item.shape[0]
    assert N_U == N_I
    E = edge_ui.shape[1]
    f32 = jnp.float32
    n_pad = _pad_dst(max(N_U, N_I))
    n_trash = n_pad - max(N_U, N_I)

    def trash_rows(k):
        return max(N_U, N_I) + (jnp.arange(k, dtype=jnp.int32) % n_trash)

    q = _C * _NW
    e_seg = ((E + q - 1) // q) * q
    per_w = e_seg // _NW
    n_chunk = per_w // _C

    def padded(row, fill):
        if e_seg == E:
            return row
        if fill is None:
            fill = jnp.zeros((e_seg - E,), jnp.int32)
        return jnp.concatenate([row, fill])

    tr_seg = trash_rows(e_seg - E) if e_seg > E else None
    src_ui = padded(edge_ui[0], None).reshape(_NW, n_chunk, _C)
    dst_ui = padded(edge_ui[1], tr_seg).reshape(_NW, n_chunk, _C)
    src_iu = padded(edge_iu[0], None).reshape(_NW, n_chunk, _C)
    dst_iu = padded(edge_iu[1], tr_seg).reshape(_NW, n_chunk, _C)

    qc = _CC * _NS
    e_cnt = ((E + qc - 1) // qc) * qc
    nc_cnt = e_cnt // _NS // _CC

    def padded_c(row):
        return jnp.concatenate([row, trash_rows(e_cnt - E)])

    dst_cnt = jnp.concatenate([
        padded_c(edge_iu[1]).reshape(_NS, nc_cnt, _CC),
        padded_c(edge_ui[1]).reshape(_NS, nc_cnt, _CC)])

    prep = pl.pallas_call(
        _prep_body,
        out_shape=[jax.ShapeDtypeStruct((N_U, H), f32),
                   jax.ShapeDtypeStruct((N_I, H), f32)],
    )
    h_u, h_i = prep(x_user, x_item,
                    p['lin_user_W'], _r2(p['lin_user_b']),
                    _r2(p['in_bn_user_g']), _r2(p['in_bn_user_b']),
                    p['lin_item_W'], _r2(p['lin_item_b']),
                    _r2(p['in_bn_item_g']), _r2(p['in_bn_item_b']))

    cnt = _make_counts(n_pad, nc_cnt)(dst_cnt)
    cnt_u, cnt_i = cnt[0], cnt[1]

    seg = _make_seg_sum(n_pad, n_chunk, H)

    layer_i = pl.pallas_call(
        _layer_body,
        out_shape=jax.ShapeDtypeStruct((N_I, H), f32),
    )
    layer_u = pl.pallas_call(
        _layer_body,
        out_shape=jax.ShapeDtypeStruct((N_U, H), f32),
    )
    for l in range(3):
        P_i = seg(h_u, src_ui, dst_ui)
        P_u = seg(h_i, src_iu, dst_iu)
        h_i_new = layer_i(P_i, cnt_i, h_i,
                          p[f'c{l}_ui_Wl'], _r2(p[f'c{l}_ui_bl']),
                          p[f'c{l}_ui_Wr'],
                          _r2(p[f'bn{l}_i_g']), _r2(p[f'bn{l}_i_b']))
        h_u_new = layer_u(P_u, cnt_u, h_u,
                          p[f'c{l}_iu_Wl'], _r2(p[f'c{l}_iu_bl']),
                          p[f'c{l}_iu_Wr'],
                          _r2(p[f'bn{l}_u_g']), _r2(p[f'bn{l}_u_b']))
        h_u, h_i = h_u_new, h_i_new

    final = pl.pallas_call(
        _final_body,
        out_shape=[jax.ShapeDtypeStruct((N_U, p['final_W'].shape[1]), f32),
                   jax.ShapeDtypeStruct((N_I, p['final_W'].shape[1]), f32)],
    )
    return final(h_u, h_i, p['final_W'], _r2(p['final_b']))


# speedup vs baseline: 1.3305x; 1.3305x over previous
import functools

import jax
import jax.numpy as jnp
from jax import lax
from jax.experimental import pallas as pl
from jax.experimental.pallas import tpu as pltpu
from jax.experimental.pallas import tpu_sc as plsc

_EPS = 1e-5

_NC = 2
_NS = 16
_NW = _NC * _NS
_C = 80
_CC = 128


def _pad_dst(n):
    q = _NS * 8
    return ((n + q - 1) // q) * q


def _chunks(total, step):
    out = []
    o = 0
    while o < total:
        s = min(step, total - o)
        out.append((o, s))
        o += s
    return out


def _seg_sum_body(n_pad, n_chunk, h, x_hbm, src_hbm, dst_hbm, out_hbm,
                  src_v, dst_v, rows_v, zero_v, acc_sh, sem):
    cid = lax.axis_index("c")
    sid = lax.axis_index("s")
    wid = cid * _NS + sid
    rows_per_sub = n_pad // _NS

    @pl.loop(0, 8)
    def _(i):
        for c in range(h // 16):
            zero_v[i, pl.ds(c * 16, 16)] = jnp.zeros((16,), jnp.float32)

    for off, sz in _chunks(rows_per_sub, 8):
        pltpu.sync_copy(zero_v.at[pl.ds(0, sz)],
                        acc_sh.at[pl.ds(sid * rows_per_sub + off, sz)])

    pltpu.sync_copy(src_hbm.at[wid], src_v)
    pltpu.sync_copy(dst_hbm.at[wid], dst_v)
    plsc.subcore_barrier()

    @pl.loop(0, n_chunk)
    def _(j):
        pltpu.async_copy(x_hbm.at[src_v.at[j]], rows_v, sem).wait()
        pltpu.sync_copy(rows_v, acc_sh.at[dst_v.at[j]], add=True)

    plsc.subcore_barrier()

    for off, sz in _chunks(rows_per_sub, 128):
        base = sid * rows_per_sub + off
        pltpu.sync_copy(acc_sh.at[pl.ds(base, sz)],
                        out_hbm.at[cid].at[pl.ds(base, sz)])


def _make_seg_sum(n_pad, n_chunk, h):
    mesh = plsc.VectorSubcoreMesh(core_axis_name="c", subcore_axis_name="s")
    return pl.kernel(
        functools.partial(_seg_sum_body, n_pad, n_chunk, h),
        out_type=jax.ShapeDtypeStruct((_NC, n_pad, h), jnp.float32),
        mesh=mesh,
        scratch_types=[
            pltpu.VMEM((n_chunk, _C), jnp.int32),
            pltpu.VMEM((n_chunk, _C), jnp.int32),
            pltpu.VMEM((_C, h), jnp.float32),
            pltpu.VMEM((8, h), jnp.float32),
            pltpu.VMEM_SHARED((n_pad, h), jnp.float32),
            pltpu.SemaphoreType.DMA,
        ],
    )


def _counts_body(n_pad, n_chunk, dst_hbm, cnt_hbm, dst_v, ones_v, acc_sh):
    cid = lax.axis_index("c")
    sid = lax.axis_index("s")
    wid = cid * _NS + sid
    rows_per_sub = n_pad // _NS

    @pl.loop(0, _CC)
    def _(i):
        for c in range(8):
            ones_v[i, pl.ds(c * 16, 16)] = jnp.zeros((16,), jnp.float32)

    for off, sz in _chunks(rows_per_sub, _CC):
        pltpu.sync_copy(ones_v.at[pl.ds(0, sz)],
                        acc_sh.at[pl.ds(sid * rows_per_sub + off, sz)])

    @pl.loop(0, _CC)
    def _(i):
        for c in range(8):
            ones_v[i, pl.ds(c * 16, 16)] = jnp.ones((16,), jnp.float32)

    pltpu.sync_copy(dst_hbm.at[wid], dst_v)
    plsc.subcore_barrier()

    @pl.loop(0, n_chunk)
    def _(j):
        pltpu.sync_copy(ones_v, acc_sh.at[dst_v.at[j]], add=True)

    plsc.subcore_barrier()

    for off, sz in _chunks(rows_per_sub, 128):
        base = sid * rows_per_sub + off
        pltpu.sync_copy(acc_sh.at[pl.ds(base, sz)],
                        cnt_hbm.at[cid].at[pl.ds(base, sz)])


def _make_counts(n_pad, n_chunk):
    mesh = plsc.VectorSubcoreMesh(core_axis_name="c", subcore_axis_name="s")
    return pl.kernel(
        functools.partial(_counts_body, n_pad, n_chunk),
        out_type=jax.ShapeDtypeStruct((_NC, n_pad, 128), jnp.float32),
        mesh=mesh,
        scratch_types=[
            pltpu.VMEM((n_chunk, _CC), jnp.int32),
            pltpu.VMEM((_CC, 128), jnp.float32),
            pltpu.VMEM_SHARED((n_pad, 128), jnp.float32),
        ],
    )


def _bn_relu(z, g, b):
    mu = jnp.mean(z, axis=0, keepdims=True)
    var = jnp.mean((z - mu) ** 2, axis=0, keepdims=True)
    return jnp.maximum((z - mu) / jnp.sqrt(var + _EPS) * g + b, 0.0)


def _prep_body(xu, xi, Wu, bu, gu, bbu, Wi, bi, gi, bbi, hu_o, hi_o):
    zu = jnp.dot(xu[...], Wu[...], preferred_element_type=jnp.float32) + bu[...]
    hu_o[...] = _bn_relu(zu, gu[...], bbu[...])
    zi = jnp.dot(xi[...], Wi[...], preferred_element_type=jnp.float32) + bi[...]
    hi_o[...] = _bn_relu(zi, gi[...], bbi[...])


def _layer_body(P, cnt, h, Wl, bl, Wr, g, bb, out):
    n = h.shape[0]
    agg = (P[0, :n] + P[1, :n]) / jnp.maximum(cnt[:n, :1], 1.0)
    z = (jnp.dot(agg, Wl[...], preferred_element_type=jnp.float32)
         + bl[...]
         + jnp.dot(h[...], Wr[...], preferred_element_type=jnp.float32))
    out[...] = _bn_relu(z, g[...], bb[...]) + h[...]


def _final_body(hu, hi, W, b, ou, oi):
    ou[...] = jnp.dot(hu[...], W[...], preferred_element_type=jnp.float32) + b[...]
    oi[...] = jnp.dot(hi[...], W[...], preferred_element_type=jnp.float32) + b[...]


def _r2(v):
    return v.reshape(1, -1)


def kernel(x_user, x_item, edge_ui, edge_iu, params):
    p = params
    N_U, H = x_user.shape
    N_I = x_item.shape[0]
    assert N_U == N_I
    E = edge_ui.shape[1]
    f32 = jnp.float32
    n_pad = _pad_dst(max(N_U, N_I))
    n_trash = n_pad - max(N_U, N_I)

    def trash_rows(k):
        return max(N_U, N_I) + (jnp.arange(k, dtype=jnp.int32) % n_trash)

    q = _C * _NW
    e_seg = ((E + q - 1) // q) * q
    per_w = e_seg // _NW
    n_chunk = per_w // _C

    def padded(row, fill):
        if e_seg == E:
            return row
        if fill is None:
            fill = jnp.zeros((e_seg - E,), jnp.int32)
        return jnp.concatenate([row, fill])

    tr_seg = trash_rows(e_seg - E) if e_seg > E else None
    src_ui = padded(edge_ui[0], None).reshape(_NW, n_chunk, _C)
    dst_ui = padded(edge_ui[1], tr_seg).reshape(_NW, n_chunk, _C)
    src_iu = padded(edge_iu[0], None).reshape(_NW, n_chunk, _C)
    dst_iu = padded(edge_iu[1], tr_seg).reshape(_NW, n_chunk, _C)

    qc = _CC * _NS
    e_cnt = ((E + qc - 1) // qc) * qc
    nc_cnt = e_cnt // _NS // _CC

    def padded_c(row):
        return jnp.concatenate([row, trash_rows(e_cnt - E)])

    dst_cnt = jnp.concatenate([
        padded_c(edge_iu[1]).reshape(_NS, nc_cnt, _CC),
        padded_c(edge_ui[1]).reshape(_NS, nc_cnt, _CC)])

    prep = pl.pallas_call(
        _prep_body,
        out_shape=[jax.ShapeDtypeStruct((N_U, H), f32),
                   jax.ShapeDtypeStruct((N_I, H), f32)],
    )
    h_u, h_i = prep(x_user, x_item,
                    p['lin_user_W'], _r2(p['lin_user_b']),
                    _r2(p['in_bn_user_g']), _r2(p['in_bn_user_b']),
                    p['lin_item_W'], _r2(p['lin_item_b']),
                    _r2(p['in_bn_item_g']), _r2(p['in_bn_item_b']))

    cnt = _make_counts(n_pad, nc_cnt)(dst_cnt)
    cnt_u, cnt_i = cnt[0], cnt[1]

    seg = _make_seg_sum(n_pad, n_chunk, H)

    layer_i = pl.pallas_call(
        _layer_body,
        out_shape=jax.ShapeDtypeStruct((N_I, H), f32),
    )
    layer_u = pl.pallas_call(
        _layer_body,
        out_shape=jax.ShapeDtypeStruct((N_U, H), f32),
    )
    for l in range(3):
        P_i = seg(h_u, src_ui, dst_ui)
        P_u = seg(h_i, src_iu, dst_iu)
        h_i_new = layer_i(P_i, cnt_i, h_i,
                          p[f'c{l}_ui_Wl'], _r2(p[f'c{l}_ui_bl']),
                          p[f'c{l}_ui_Wr'],
                          _r2(p[f'bn{l}_i_g']), _r2(p[f'bn{l}_i_b']))
        h_u_new = layer_u(P_u, cnt_u, h_u,
                          p[f'c{l}_iu_Wl'], _r2(p[f'c{l}_iu_bl']),
                          p[f'c{l}_iu_Wr'],
                          _r2(p[f'bn{l}_u_g']), _r2(p[f'bn{l}_u_b']))
        h_u, h_i = h_u_new, h_i_new

    final = pl.pallas_call(
        _final_body,
        out_shape=[jax.ShapeDtypeStruct((N_U, p['final_W'].shape[1]), f32),
                   jax.ShapeDtypeStruct((N_I, p['final_W'].shape[1]), f32)],
    )
    return final(h_u, h_i, p['final_W'], _r2(p['final_b']))
